# baseline (speedup 1.0000x reference)
import functools

import jax
import jax.numpy as jnp
from jax import lax
from jax.experimental import pallas as pl
from jax.experimental.pallas import tpu as pltpu
from jax.experimental.pallas import tpu_sc as plsc

_info = plsc.get_sparse_core_info()
_NC, _NS, _L = _info.num_cores, _info.num_subcores, _info.num_lanes
_NW = _NC * _NS
_NSLOT = 5


def _make_sc_kernel(V, D, E):
    assert E % _NW == 0
    e_w = E // _NW
    B = 80
    assert e_w % B == 0 and B % _L == 0 and D % (2 * _L) == 0
    n_chunks = e_w // B
    assert n_chunks % _NSLOT == 0
    groups = B // _L
    Dw = D // 2
    n_sl = Dw // _L

    mesh = plsc.VectorSubcoreMesh(core_axis_name="c", subcore_axis_name="s")

    @functools.partial(
        pl.kernel,
        mesh=mesh,
        compiler_params=pltpu.CompilerParams(
            needs_layout_passes=False, use_tc_tiling_on_sc=False),
        out_type=jax.ShapeDtypeStruct((E,), jnp.float32),
        scratch_types=(
            [pltpu.VMEM((B, Dw), jnp.int32)] * (2 * _NSLOT)
            + [
                pltpu.VMEM((e_w,), jnp.int32),
                pltpu.VMEM((e_w,), jnp.int32),
                pltpu.VMEM((D,), jnp.float32),
                pltpu.VMEM((e_w,), jnp.float32),
            ]
            + [pltpu.SemaphoreType.DMA] * (2 * _NSLOT)
        ),
    )
    def k(table_hbm, ridx_hbm, cidx_hbm, w_hbm, out_hbm, *rest):
        rbufs = rest[0:_NSLOT]
        cbufs = rest[_NSLOT:2 * _NSLOT]
        ridx_v, cidx_v, w_v, out_v = rest[2 * _NSLOT:2 * _NSLOT + 4]
        sems_r = rest[2 * _NSLOT + 4:3 * _NSLOT + 4]
        sems_c = rest[3 * _NSLOT + 4:4 * _NSLOT + 4]

        wid = lax.axis_index("s") * _NC + lax.axis_index("c")
        base = wid * e_w
        pltpu.sync_copy(ridx_hbm.at[pl.ds(base, e_w)], ridx_v)
        pltpu.sync_copy(cidx_hbm.at[pl.ds(base, e_w)], cidx_v)
        pltpu.sync_copy(w_hbm, w_v)

        lanes = lax.iota(jnp.int32, _L)
        zero = jnp.zeros((_L,), jnp.float32)

        def issue(ch, s):
            off = ch * B
            pltpu.async_copy(
                table_hbm.at[ridx_v.at[pl.ds(off, B)]], rbufs[s], sems_r[s])
            pltpu.async_copy(
                table_hbm.at[cidx_v.at[pl.ds(off, B)]], cbufs[s], sems_c[s])

        def wait(s):
            pltpu.make_async_copy(
                table_hbm.at[ridx_v.at[pl.ds(0, B)]],
                rbufs[s], sems_r[s]).wait()
            pltpu.make_async_copy(
                table_hbm.at[cidx_v.at[pl.ds(0, B)]],
                cbufs[s], sems_c[s]).wait()

        we_regs = [w_v[pl.ds(i * _L, _L)] for i in range(n_sl)]
        wo_regs = [w_v[pl.ds(Dw + i * _L, _L)] for i in range(n_sl)]

        def compute(ch, s):
            off = ch * B
            rbuf, cbuf = rbufs[s], cbufs[s]

            def group_body(eb, _):
                def quad_body(q, res):
                    for kk in range(4):
                        j = q * 4 + kk
                        e = eb * _L + j
                        acc_e = zero
                        acc_o = zero
                        for i in range(n_sl):
                            rv = plsc.bitcast(
                                rbuf[e, pl.ds(i * _L, _L)], jnp.bfloat16)
                            cv = plsc.bitcast(
                                cbuf[e, pl.ds(i * _L, _L)], jnp.bfloat16)
                            da, db = plsc.unpack(
                                jnp.abs(rv - cv),
                                format=plsc.PackFormat.INTERLEAVED)
                            acc_e = acc_e + da * we_regs[i]
                            acc_o = acc_o + db * wo_regs[i]
                        s_ = jnp.sum(acc_e + acc_o)
                        res = jnp.where(lanes == j, s_, res)
                    return res

                res = lax.fori_loop(0, 4, quad_body, zero)
                sig = 1.0 / (1.0 + jnp.exp(-res))
                out_v[pl.ds(off + eb * _L, _L)] = sig
                return 0

            lax.fori_loop(0, groups, group_body, 0)

        for s in range(_NSLOT - 1):
            issue(s, s)

        def body(p, _):
            for s in range(_NSLOT):
                ch = _NSLOT * p + s
                wait(s)
                nxt = ch + _NSLOT - 1

                @pl.when(nxt < n_chunks)
                def _():
                    issue(nxt, (s + _NSLOT - 1) % _NSLOT)

                compute(ch, s)
            return 0

        lax.fori_loop(0, n_chunks // _NSLOT, body, 0)
        pltpu.sync_copy(out_v, out_hbm.at[pl.ds(base, e_w)])

    return k


def kernel(inputs, r_indices, c_indices, weights):
    V, D = inputs.shape
    E = r_indices.shape[0]
    r32 = r_indices.astype(jnp.int32)
    c32 = c_indices.astype(jnp.int32)
    t16 = inputs.astype(jnp.bfloat16).reshape(V, D // 2, 2)
    t32 = jax.lax.bitcast_convert_type(t16, jnp.int32)
    w = weights.reshape(-1).astype(jnp.float32)
    w_de = jnp.concatenate([w[0::2], w[1::2]])
    k = _make_sc_kernel(V, D, E)
    return k(t32, r32, c32, w_de)
